# Initial kernel scaffold; baseline (speedup 1.0000x reference)
import jax
import jax.numpy as jnp
from jax.experimental import pallas as pl


def kernel(x, table):
    raise NotImplementedError("write your pallas kernel here")


# speedup vs baseline: 1.8250x; 1.8250x over previous
import functools

import jax
import jax.numpy as jnp
from jax import lax
from jax.experimental import pallas as pl
from jax.experimental.pallas import tpu as pltpu
from jax.experimental.pallas import tpu_sc as plsc

BATCH = 4
SEQ = 2048
N_TOKENS = 8192
D = 8192

NC = 2
NS = 16
NW = NC * NS
B_TOTAL = BATCH * SEQ
BPW = B_TOTAL // NW
CH = 8
NCHUNK = BPW // CH

_mesh = plsc.VectorSubcoreMesh(core_axis_name="c", subcore_axis_name="s")


@functools.partial(
    pl.kernel,
    mesh=_mesh,
    out_type=jax.ShapeDtypeStruct((B_TOTAL, D), jnp.float32),
    scratch_types=[
        pltpu.VMEM((NCHUNK, CH), jnp.int32),
        pltpu.VMEM((CH, D), jnp.float32),
        pltpu.SemaphoreType.DMA,
    ],
)
def _sc_gather(x_hbm, table_hbm, out_hbm, idx_v, rows_v, sem):
    wid = lax.axis_index("s") * NC + lax.axis_index("c")
    base = wid * BPW
    pltpu.sync_copy(x_hbm.at[wid], idx_v)

    def chunk(i, carry):
        pltpu.async_copy(table_hbm.at[idx_v.at[i]], rows_v, sem).wait()
        pltpu.sync_copy(rows_v, out_hbm.at[pl.ds(base + i * CH, CH)])
        return carry

    lax.fori_loop(0, NCHUNK, chunk, 0)


def kernel(x, table):
    idx = x.reshape(NW, NCHUNK, CH).astype(jnp.int32)
    out = _sc_gather(idx, table)
    return out.reshape(BATCH, SEQ, D)


# speedup vs baseline: 1.9770x; 1.0833x over previous
import functools

import jax
import jax.numpy as jnp
from jax import lax
from jax.experimental import pallas as pl
from jax.experimental.pallas import tpu as pltpu
from jax.experimental.pallas import tpu_sc as plsc

BATCH = 4
SEQ = 2048
N_TOKENS = 8192
D = 8192

NC = 2
NS = 16
NW = NC * NS
B_TOTAL = BATCH * SEQ
BPW = B_TOTAL // NW
CH = 4
NCHUNK = BPW // CH
NG = NCHUNK // 2

_mesh = plsc.VectorSubcoreMesh(core_axis_name="c", subcore_axis_name="s")


@functools.partial(
    pl.kernel,
    mesh=_mesh,
    out_type=jax.ShapeDtypeStruct((B_TOTAL, D), jnp.float32),
    scratch_types=[
        pltpu.VMEM((NCHUNK, CH), jnp.int32),
        pltpu.VMEM((2, CH, D), jnp.float32),
        pltpu.SemaphoreType.DMA,
        pltpu.SemaphoreType.DMA,
        pltpu.SemaphoreType.DMA,
        pltpu.SemaphoreType.DMA,
    ],
)
def _sc_gather(x_hbm, table_hbm, out_hbm, idx_v, rows_v, gsem0, gsem1,
               ssem0, ssem1):
    wid = lax.axis_index("s") * NC + lax.axis_index("c")
    base = wid * BPW
    gsems = (gsem0, gsem1)
    ssems = (ssem0, ssem1)
    pltpu.sync_copy(x_hbm.at[wid], idx_v)

    def g_start(slot, i):
        pltpu.async_copy(table_hbm.at[idx_v.at[i]], rows_v.at[slot],
                         gsems[slot])

    def g_wait(slot):
        pltpu.make_async_copy(table_hbm.at[pl.ds(0, CH)], rows_v.at[slot],
                              gsems[slot]).wait()

    def s_start(slot, i):
        pltpu.async_copy(rows_v.at[slot], out_hbm.at[pl.ds(base + i * CH, CH)],
                         ssems[slot])

    def s_wait(slot):
        pltpu.make_async_copy(rows_v.at[slot], out_hbm.at[pl.ds(0, CH)],
                              ssems[slot]).wait()

    def pair(g, first=False, last=False):
        i0 = 2 * g
        i1 = i0 + 1
        if not first:
            s_wait(1)
        g_start(1, i1)
        g_wait(0)
        s_start(0, i0)
        s_wait(0)
        if not last:
            g_start(0, i1 + 1)
        g_wait(1)
        s_start(1, i1)

    g_start(0, 0)
    pair(0, first=True)
    lax.fori_loop(1, NG - 1, lambda g, c: (pair(g), c)[1], 0)
    pair(NG - 1, last=True)
    s_wait(1)


def kernel(x, table):
    idx = x.reshape(NW, NCHUNK, CH).astype(jnp.int32)
    out = _sc_gather(idx, table)
    return out.reshape(BATCH, SEQ, D)
